# baseline (speedup 1.0000x reference)
import jax
import jax.numpy as jnp
from jax.experimental import pallas as pl
from jax.experimental.pallas import tpu as pltpu

B = 4096
IN_DIM = 512
H = 512
E = 8
TOP_K = 2


def _fused_kernel(x_ref, W_in_ref, b_in_ref, Wg_ref, bg_ref,
                  W1_ref, b1_ref, W2_ref, b2_ref,
                  Wo1_ref, bo1_ref, Wo2_ref, bo2_ref, out_ref,
                  h_s, acc_s, i1_s, i2_s, g1_s, g2_s):
    e = pl.program_id(0)

    @pl.when(e == 0)
    def _gate():
        h = jnp.dot(x_ref[...], W_in_ref[...],
                    preferred_element_type=jnp.float32)
        h = h + b_in_ref[...]
        h_s[...] = h
        gl = jnp.dot(h, Wg_ref[...], preferred_element_type=jnp.float32)
        gl = gl + bg_ref[...]
        eids = jax.lax.broadcasted_iota(jnp.int32, gl.shape, 1)
        l1 = jnp.max(gl, axis=-1, keepdims=True)
        i1 = jnp.argmax(gl, axis=-1).reshape(B, 1)
        masked = jnp.where(eids == i1, -jnp.inf, gl)
        l2 = jnp.max(masked, axis=-1, keepdims=True)
        i2 = jnp.argmax(masked, axis=-1).reshape(B, 1)
        r = jnp.exp(l2 - l1)
        g1 = 1.0 / (1.0 + r)
        i1_s[...] = i1
        i2_s[...] = i2
        g1_s[...] = g1
        g2_s[...] = 1.0 - g1
        acc_s[...] = jnp.zeros((B, H), jnp.float32)

    h1 = jnp.dot(h_s[...], W1_ref[0], preferred_element_type=jnp.float32)
    h1 = jnp.maximum(h1 + b1_ref[0], 0.0)
    o = jnp.dot(h1, W2_ref[0], preferred_element_type=jnp.float32)
    o = o + b2_ref[0]
    coef = (jnp.where(i1_s[...] == e, g1_s[...], 0.0) +
            jnp.where(i2_s[...] == e, g2_s[...], 0.0))
    acc_s[...] = acc_s[...] + coef * o

    @pl.when(e == E - 1)
    def _head():
        z = jnp.dot(acc_s[...], Wo1_ref[...],
                    preferred_element_type=jnp.float32)
        z = jnp.maximum(z + bo1_ref[...], 0.0)
        p = jnp.sum(z * Wo2_ref[...], axis=-1, keepdims=True)
        out_ref[...] = p + bo2_ref[...]


def kernel(x, W_in, b_in, Wg, bg, W1, b1, W2, b2, Wo1, bo1, Wo2, bo2):
    def full(*shape):
        return pl.BlockSpec(shape, lambda e: (0,) * len(shape))

    out = pl.pallas_call(
        _fused_kernel,
        grid=(E,),
        in_specs=[
            full(B, IN_DIM),
            full(IN_DIM, H),
            full(1, H),
            full(H, E),
            full(1, E),
            pl.BlockSpec((1, H, H), lambda e: (e, 0, 0)),
            pl.BlockSpec((1, 1, H), lambda e: (e, 0, 0)),
            pl.BlockSpec((1, H, H), lambda e: (e, 0, 0)),
            pl.BlockSpec((1, 1, H), lambda e: (e, 0, 0)),
            full(H, H // 2),
            full(1, H // 2),
            full(1, H // 2),
            full(1, 1),
        ],
        out_specs=full(B, 1),
        out_shape=jax.ShapeDtypeStruct((B, 1), jnp.float32),
        scratch_shapes=[
            pltpu.VMEM((B, H), jnp.float32),
            pltpu.VMEM((B, H), jnp.float32),
            pltpu.VMEM((B, 1), jnp.int32),
            pltpu.VMEM((B, 1), jnp.int32),
            pltpu.VMEM((B, 1), jnp.float32),
            pltpu.VMEM((B, 1), jnp.float32),
        ],
        compiler_params=pltpu.CompilerParams(
            dimension_semantics=("arbitrary",),
        ),
    )(x, W_in, b_in.reshape(1, H), Wg, bg.reshape(1, E),
      W1, b1.reshape(E, 1, H), W2, b2.reshape(E, 1, H),
      Wo1, bo1.reshape(1, H // 2), Wo2.reshape(1, H // 2),
      bo2.reshape(1, 1))
    return out


# speedup vs baseline: 1.1668x; 1.1668x over previous
import jax
import jax.numpy as jnp
from jax.experimental import pallas as pl
from jax.experimental.pallas import tpu as pltpu

B = 4096
IN_DIM = 512
H = 512
E = 8
TOP_K = 2
BT = 1024


def _fused_kernel(x_ref, W_in_ref, b_in_ref, Wg_ref, bg_ref,
                  W1_ref, b1_ref, W2_ref, b2_ref,
                  Wo1_ref, bo1_ref, Wo2_ref, bo2_ref, out_ref):
    x = x_ref[...]
    h = jnp.dot(x, W_in_ref[...], preferred_element_type=jnp.float32)
    h = h + b_in_ref[...]
    gl = jnp.dot(h, Wg_ref[...], preferred_element_type=jnp.float32)
    gl = gl + bg_ref[...]

    eids = jax.lax.broadcasted_iota(jnp.int32, gl.shape, 1)
    l1 = jnp.max(gl, axis=-1)
    i1 = jnp.argmax(gl, axis=-1)
    masked = jnp.where(eids == i1[:, None], -jnp.inf, gl)
    l2 = jnp.max(masked, axis=-1)
    i2 = jnp.argmax(masked, axis=-1)
    r = jnp.exp(l2 - l1)
    g1 = 1.0 / (1.0 + r)
    g2 = 1.0 - g1

    acc = jnp.zeros((BT, H), jnp.float32)
    for e in range(E):
        h1 = jnp.dot(h, W1_ref[e], preferred_element_type=jnp.float32)
        h1 = jnp.maximum(h1 + b1_ref[e], 0.0)
        o = jnp.dot(h1, W2_ref[e], preferred_element_type=jnp.float32)
        o = o + b2_ref[e]
        coef = jnp.where(i1 == e, g1, 0.0) + jnp.where(i2 == e, g2, 0.0)
        acc = acc + coef[:, None] * o

    z = jnp.dot(acc, Wo1_ref[...], preferred_element_type=jnp.float32)
    z = jnp.maximum(z + bo1_ref[...], 0.0)
    p = jnp.sum(z * Wo2_ref[...], axis=-1, keepdims=True)
    out_ref[...] = p + bo2_ref[...]


def kernel(x, W_in, b_in, Wg, bg, W1, b1, W2, b2, Wo1, bo1, Wo2, bo2):
    grid = (B // BT,)

    def full(*shape):
        return pl.BlockSpec(shape, lambda i: (0,) * len(shape))

    out = pl.pallas_call(
        _fused_kernel,
        grid=grid,
        in_specs=[
            pl.BlockSpec((BT, IN_DIM), lambda i: (i, 0)),
            full(IN_DIM, H),
            full(1, H),
            full(H, E),
            full(1, E),
            full(E, H, H),
            full(E, H),
            full(E, H, H),
            full(E, H),
            full(H, H // 2),
            full(1, H // 2),
            full(1, H // 2),
            full(1, 1),
        ],
        out_specs=pl.BlockSpec((BT, 1), lambda i: (i, 0)),
        out_shape=jax.ShapeDtypeStruct((B, 1), jnp.float32),
        compiler_params=pltpu.CompilerParams(
            dimension_semantics=("parallel",),
        ),
    )(x, W_in, b_in.reshape(1, H), Wg, bg.reshape(1, E),
      W1, b1, W2, b2,
      Wo1, bo1.reshape(1, H // 2), Wo2.reshape(1, H // 2),
      bo2.reshape(1, 1))
    return out


# speedup vs baseline: 1.1685x; 1.0015x over previous
import jax
import jax.numpy as jnp
from jax.experimental import pallas as pl
from jax.experimental.pallas import tpu as pltpu

B = 4096
IN_DIM = 512
H = 512
E = 8
TOP_K = 2
BT = 1024


def _fused_kernel(x_ref, W_in_ref, b_in_ref, Wg_ref, bg_ref,
                  W1_ref, b1_ref, W2_ref, b2_ref,
                  Wo1_ref, bo1_ref, Wo2_ref, bo2_ref, out_ref):
    x = x_ref[...]
    h = jnp.dot(x, W_in_ref[...], preferred_element_type=jnp.float32)
    h = h + b_in_ref[...]
    gl = jnp.dot(h, Wg_ref[...], preferred_element_type=jnp.float32)
    gl = gl + bg_ref[...]

    eids = jax.lax.broadcasted_iota(jnp.int32, gl.shape, 1)
    l1 = jnp.max(gl, axis=-1)
    i1 = jnp.argmax(gl, axis=-1)
    masked = jnp.where(eids == i1[:, None], -jnp.inf, gl)
    l2 = jnp.max(masked, axis=-1)
    i2 = jnp.argmax(masked, axis=-1)
    r = jnp.exp(l2 - l1)
    g1 = 1.0 / (1.0 + r)
    g2 = 1.0 - g1

    acc = jnp.zeros((BT, H), jnp.float32)
    for e in range(E):
        h1 = jnp.dot(h, W1_ref[e], preferred_element_type=jnp.float32)
        h1 = jnp.maximum(h1 + b1_ref[e], 0.0)
        o = jnp.dot(h1, W2_ref[e], preferred_element_type=jnp.float32)
        o = o + b2_ref[e]
        coef = jnp.where(i1 == e, g1, 0.0) + jnp.where(i2 == e, g2, 0.0)
        acc = acc + coef[:, None] * o

    z = jnp.dot(acc, Wo1_ref[...], preferred_element_type=jnp.float32)
    z = jnp.maximum(z + bo1_ref[...], 0.0)
    p = jnp.sum(z * Wo2_ref[...], axis=-1, keepdims=True)
    out_ref[...] = p + bo2_ref[...]


def kernel(x, W_in, b_in, Wg, bg, W1, b1, W2, b2, Wo1, bo1, Wo2, bo2):
    grid = (B // BT,)

    def full(*shape):
        return pl.BlockSpec(shape, lambda i: (0,) * len(shape))

    out = pl.pallas_call(
        _fused_kernel,
        grid=grid,
        in_specs=[
            pl.BlockSpec((BT, IN_DIM), lambda i: (i, 0)),
            full(IN_DIM, H),
            full(1, H),
            full(H, E),
            full(1, E),
            full(E, H, H),
            full(E, H),
            full(E, H, H),
            full(E, H),
            full(H, H // 2),
            full(1, H // 2),
            full(1, H // 2),
            full(1, 1),
        ],
        out_specs=pl.BlockSpec((BT, 1), lambda i: (i, 0)),
        out_shape=jax.ShapeDtypeStruct((B, 1), jnp.float32),
        compiler_params=pltpu.CompilerParams(
            dimension_semantics=("arbitrary",),
        ),
    )(x, W_in, b_in.reshape(1, H), Wg, bg.reshape(1, E),
      W1, b1, W2, b2,
      Wo1, bo1.reshape(1, H // 2), Wo2.reshape(1, H // 2),
      bo2.reshape(1, 1))
    return out
